# baseline (speedup 1.0000x reference)
import functools

import jax
import jax.numpy as jnp
from jax import lax
from jax.experimental import pallas as pl
from jax.experimental.pallas import tpu as pltpu
from jax.experimental.pallas import tpu_sc as plsc

BATCH = 16384
DIM = 32

_info = plsc.get_sparse_core_info()
_NC, _NS, _L = _info.num_cores, _info.num_subcores, _info.num_lanes
_NW = _NC * _NS
_BPW = BATCH // _NW
_GROUPS = _BPW // _L


def _sc_body(uidx_hbm, iidx_hbm, utab_hbm, itab_hbm, out_hbm,
             uidx_v, iidx_v, urows_v, irows_v, out_v, sem_u, sem_i):
    wid = lax.axis_index("s") * _NC + lax.axis_index("c")
    base = wid * _BPW

    pltpu.sync_copy(uidx_hbm.at[pl.ds(base, _BPW)], uidx_v)
    pltpu.sync_copy(iidx_hbm.at[pl.ds(base, _BPW)], iidx_v)

    cu = pltpu.async_copy(utab_hbm.at[uidx_v], urows_v, sem_u)
    ci = pltpu.async_copy(itab_hbm.at[iidx_v], irows_v, sem_i)
    cu.wait()
    ci.wait()

    lane = lax.iota(jnp.int32, _L)

    def group(g, carry):
        rows = g * _L + lane
        acc = jnp.zeros((_L,), jnp.float32)
        for j in range(DIM):
            col = jnp.full((_L,), j, dtype=jnp.int32)
            uu = plsc.load_gather(urows_v, [rows, col])
            vv = plsc.load_gather(irows_v, [rows, col])
            acc = acc + uu * vv
        out_v[pl.ds(g * _L, _L)] = acc
        return carry

    lax.fori_loop(0, _GROUPS, group, 0)

    pltpu.sync_copy(out_v, out_hbm.at[pl.ds(base, _BPW)])


@jax.jit
def _run(user_indices, item_indices, user_table, item_table):
    mesh = plsc.VectorSubcoreMesh(core_axis_name="c", subcore_axis_name="s")
    f = functools.partial(
        pl.kernel,
        out_type=jax.ShapeDtypeStruct((BATCH,), jnp.float32),
        mesh=mesh,
        compiler_params=pltpu.CompilerParams(
            needs_layout_passes=False, use_tc_tiling_on_sc=False),
        scratch_types=[
            pltpu.VMEM((_BPW,), jnp.int32),
            pltpu.VMEM((_BPW,), jnp.int32),
            pltpu.VMEM((_BPW, DIM), jnp.float32),
            pltpu.VMEM((_BPW, DIM), jnp.float32),
            pltpu.VMEM((_BPW,), jnp.float32),
            pltpu.SemaphoreType.DMA,
            pltpu.SemaphoreType.DMA,
        ],
    )(_sc_body)
    return f(user_indices, item_indices, user_table, item_table)


def kernel(user_indices, item_indices, user_table, item_table):
    return _run(user_indices.astype(jnp.int32), item_indices.astype(jnp.int32),
                user_table, item_table)


# speedup vs baseline: 3.7554x; 3.7554x over previous
import functools

import jax
import jax.numpy as jnp
from jax import lax
from jax.experimental import pallas as pl
from jax.experimental.pallas import tpu as pltpu
from jax.experimental.pallas import tpu_sc as plsc

BATCH = 16384
DIM = 32

_info = plsc.get_sparse_core_info()
_NC, _NS, _L = _info.num_cores, _info.num_subcores, _info.num_lanes
_NW = _NC * _NS
_BPW = BATCH // _NW
_WAVES = _BPW // _L


def _sc_body(uidx_hbm, iidx_hbm, utabT, itabT, out_hbm,
             uidx_v, iidx_v, slabs, uvex, out_v, sem):
    wid = lax.axis_index("s") * _NC + lax.axis_index("c")
    base = wid * _BPW

    pltpu.sync_copy(uidx_hbm.at[pl.ds(base, _BPW)], uidx_v)
    pltpu.sync_copy(iidx_hbm.at[pl.ds(base, _BPW)], iidx_v)

    lane = lax.iota(jnp.int32, _L)

    def wave(w, carry):
        s = w * _L
        vu = uidx_v[pl.ds(s, _L)]
        vi = iidx_v[pl.ds(s, _L)]
        cvu = vu & 127
        cvi = vi & 127

        copies = []
        for m in range(_L):
            cb = (vu[m] >> 7) * 128
            copies.append(pltpu.async_copy(
                utabT.at[:, pl.ds(cb, 128)],
                slabs.at[pl.ds(m * DIM, DIM), :], sem))
        for c in copies:
            c.wait()

        for j in range(DIM):
            rows = lane * DIM + j
            uvex[pl.ds(j * _L, _L)] = plsc.load_gather(slabs, [rows, cvu])

        copies = []
        for m in range(_L):
            cb = (vi[m] >> 7) * 128
            copies.append(pltpu.async_copy(
                itabT.at[:, pl.ds(cb, 128)],
                slabs.at[pl.ds(m * DIM, DIM), :], sem))
        for c in copies:
            c.wait()

        acc = jnp.zeros((_L,), jnp.float32)
        for j in range(DIM):
            rows = lane * DIM + j
            ig = plsc.load_gather(slabs, [rows, cvi])
            acc = acc + uvex[pl.ds(j * _L, _L)] * ig
        out_v[pl.ds(s, _L)] = acc
        return carry

    lax.fori_loop(0, _WAVES, wave, 0)

    pltpu.sync_copy(out_v, out_hbm.at[pl.ds(base, _BPW)])


@jax.jit
def _run(user_indices, item_indices, user_table, item_table):
    mesh = plsc.VectorSubcoreMesh(core_axis_name="c", subcore_axis_name="s")
    f = functools.partial(
        pl.kernel,
        out_type=jax.ShapeDtypeStruct((BATCH,), jnp.float32),
        mesh=mesh,
        compiler_params=pltpu.CompilerParams(needs_layout_passes=False),
        scratch_types=[
            pltpu.VMEM((_BPW,), jnp.int32),
            pltpu.VMEM((_BPW,), jnp.int32),
            pltpu.VMEM((_L * DIM, 128), jnp.float32),
            pltpu.VMEM((DIM * _L,), jnp.float32),
            pltpu.VMEM((_BPW,), jnp.float32),
            pltpu.SemaphoreType.DMA,
        ],
    )(_sc_body)
    return f(user_indices, item_indices, user_table.T, item_table.T)


def kernel(user_indices, item_indices, user_table, item_table):
    return _run(user_indices.astype(jnp.int32), item_indices.astype(jnp.int32),
                user_table, item_table)
